# baseline (speedup 1.0000x reference)
import functools

import jax
import jax.numpy as jnp
from jax import lax
from jax.experimental import pallas as pl
from jax.experimental.pallas import tpu as pltpu
from jax.experimental.pallas import tpu_sc as plsc

_NCORES = 2
_NSUB = 16
_NTILES = _NCORES * _NSUB
_CHUNK = 128
_NB = 4
_DEGW = 8

_SLOW_CORE = 1
_SLOW_FRAC = 0.25


def _chunk_split(total_chunks):
    ns = int(total_chunks * _SLOW_FRAC) // (_NSUB * _NB) * _NB
    nf = total_chunks // _NSUB - ns
    return ns, nf


def _pad_to(v, m):
    return (v + m - 1) // m * m


def _relu(v):
    return jnp.maximum(v, 0.0)


def _dot(a, b):
    return jnp.dot(a, b, preferred_element_type=jnp.float32)


def _mesh():
    return plsc.VectorSubcoreMesh(core_axis_name="c", subcore_axis_name="s",
                                  num_cores=_NCORES, num_subcores=_NSUB)


@functools.lru_cache(maxsize=None)
def _deg_kernel(NP, Ep):
    ns, nf = _chunk_split(Ep // _CHUNK)
    rpt = NP // _NSUB

    @functools.partial(
        pl.kernel,
        out_type=jax.ShapeDtypeStruct((_NCORES, NP, _DEGW), jnp.float32),
        mesh=_mesh(),
        scratch_types=[
            pltpu.VMEM((nf, _CHUNK), jnp.int32),
            pltpu.VMEM((_CHUNK, _DEGW), jnp.float32),
            pltpu.VMEM_SHARED((NP, _DEGW), jnp.float32),
        ] + [pltpu.SemaphoreType.DMA] * _NB,
        compiler_params=pltpu.CompilerParams(use_tc_tiling_on_sc=False),
    )
    def deg_kernel(dst_hbm, ones_hbm, zeros_hbm, out_hbm, dst_i, ones_v,
                   acc_sh, *sems):
        c = lax.axis_index("c")
        s = lax.axis_index("s")
        nch = jnp.where(c == _SLOW_CORE, ns, nf)
        base = jnp.where(c == _SLOW_CORE, s * ns, _NSUB * ns + s * nf)
        pltpu.sync_copy(dst_hbm.at[pl.ds(base, nf)], dst_i)
        pltpu.sync_copy(ones_hbm, ones_v)
        pltpu.sync_copy(zeros_hbm.at[pl.ds(s * rpt, rpt)],
                        acc_sh.at[pl.ds(s * rpt, rpt)])
        plsc.subcore_barrier()

        for b in range(_NB):
            pltpu.async_copy(ones_v, acc_sh.at[dst_i.at[b]], sems[b],
                             add=True)

        def group(jj, carry):
            j0 = jj * _NB
            for b in range(_NB):
                j = j0 + b
                pltpu.make_async_copy(ones_v, acc_sh.at[dst_i.at[j]],
                                      sems[b]).wait()
                jn = j + _NB

                @pl.when(jn < nch)
                def _():
                    pltpu.async_copy(ones_v, acc_sh.at[dst_i.at[jn]],
                                     sems[b], add=True)
            return carry

        lax.fori_loop(0, nch // _NB, group, 0)
        plsc.subcore_barrier()
        pltpu.sync_copy(acc_sh.at[pl.ds(s * rpt, rpt)],
                        out_hbm.at[c, pl.ds(s * rpt, rpt)])

    return deg_kernel


@functools.lru_cache(maxsize=None)
def _prop_kernel(NP, C, Ep):
    ns, nf = _chunk_split(Ep // _CHUNK)
    rpt = NP // _NSUB

    @functools.partial(
        pl.kernel,
        out_type=jax.ShapeDtypeStruct((_NCORES, NP, C), jnp.float32),
        mesh=_mesh(),
        scratch_types=[
            pltpu.VMEM((nf, _CHUNK), jnp.int32),
            pltpu.VMEM((nf, _CHUNK), jnp.int32),
        ] + [pltpu.VMEM((_CHUNK, C), jnp.float32)] * _NB + [
            pltpu.VMEM_SHARED((NP, C), jnp.float32),
        ] + [pltpu.SemaphoreType.DMA] * _NB,
        compiler_params=pltpu.CompilerParams(use_tc_tiling_on_sc=False),
    )
    def prop_kernel(u_hbm, src_hbm, dst_hbm, zeros_hbm, out_hbm,
                    src_i, dst_i, *rest):
        rows = rest[:_NB]
        acc_sh = rest[_NB]
        sems = rest[_NB + 1:]
        c = lax.axis_index("c")
        s = lax.axis_index("s")
        nch = jnp.where(c == _SLOW_CORE, ns, nf)
        base = jnp.where(c == _SLOW_CORE, s * ns, _NSUB * ns + s * nf)
        pltpu.sync_copy(src_hbm.at[pl.ds(base, nf)], src_i)
        pltpu.sync_copy(dst_hbm.at[pl.ds(base, nf)], dst_i)
        for b in range(_NB):
            pltpu.async_copy(u_hbm.at[src_i.at[b]], rows[b], sems[b])
        pltpu.sync_copy(zeros_hbm.at[pl.ds(s * rpt, rpt)],
                        acc_sh.at[pl.ds(s * rpt, rpt)])
        plsc.subcore_barrier()

        def group(jj, carry):
            j0 = jj * _NB
            for b in range(_NB):
                j = j0 + b
                pltpu.make_async_copy(u_hbm.at[src_i.at[j]], rows[b],
                                      sems[b]).wait()
                pltpu.sync_copy(rows[b], acc_sh.at[dst_i.at[j]], add=True)
                jn = j + _NB

                @pl.when(jn < nch)
                def _():
                    pltpu.async_copy(u_hbm.at[src_i.at[jn]], rows[b], sems[b])
            return carry

        lax.fori_loop(0, nch // _NB, group, 0)
        plsc.subcore_barrier()
        pltpu.sync_copy(acc_sh.at[pl.ds(s * rpt, rpt)],
                        out_hbm.at[c, pl.ds(s * rpt, rpt)])

    return prop_kernel


def _full_spec(*shape):
    nd = len(shape)
    return pl.BlockSpec(shape, lambda i, _nd=nd: (0,) * _nd)


@functools.lru_cache(maxsize=None)
def _pre_kernel(NP, D, C, BR):
    def body(x_ref, wp_ref, bp_ref, w1_ref, b1_ref, w2_ref, b2_ref,
             ni_ref, h0_ref):
        h = _relu(_dot(x_ref[...], wp_ref[...]) + bp_ref[...])
        ni_ref[...] = _relu(_dot(h, w1_ref[...]) + b1_ref[...])
        h0_ref[...] = _relu(_dot(h, w2_ref[...]) + b2_ref[...])

    return pl.pallas_call(
        body,
        grid=(NP // BR,),
        in_specs=[
            pl.BlockSpec((BR, D), lambda i: (i, 0)),
            _full_spec(D, C),
            _full_spec(1, C),
            _full_spec(C, C),
            _full_spec(1, C),
            _full_spec(C, C),
            _full_spec(1, C),
        ],
        out_specs=[pl.BlockSpec((BR, C), lambda i: (i, 0)),
                   pl.BlockSpec((BR, C), lambda i: (i, 0))],
        out_shape=[jax.ShapeDtypeStruct((NP, C), jnp.float32),
                   jax.ShapeDtypeStruct((NP, C), jnp.float32)],
    )


@functools.lru_cache(maxsize=None)
def _u_kernel(NP, C, BR):
    def body(h_ref, dp_ref, u_ref):
        dpa = dp_ref[...]
        dis = lax.rsqrt(dpa[0, :, 0:1] + dpa[1, :, 0:1] + 1.0)
        u_ref[...] = dis * h_ref[...]

    return pl.pallas_call(
        body,
        grid=(NP // BR,),
        in_specs=[
            pl.BlockSpec((BR, C), lambda i: (i, 0)),
            pl.BlockSpec((2, BR, _DEGW), lambda i: (0, i, 0)),
        ],
        out_specs=pl.BlockSpec((BR, C), lambda i: (i, 0)),
        out_shape=jax.ShapeDtypeStruct((NP, C), jnp.float32),
    )


@functools.lru_cache(maxsize=None)
def _layer_kernel(NP, C, BR, last):
    def body(sp_ref, u_ref, ni_ref, dp_ref, wg_ref, bg_ref, wfa_ref, wfb_ref,
             bf_ref, *rest):
        dpa = dp_ref[...]
        dis = lax.rsqrt(dpa[0, :, 0:1] + dpa[1, :, 0:1] + 1.0)
        spa = sp_ref[...]
        u = u_ref[...]
        agg = dis * (spa[0] + spa[1] + u)
        hg = _relu(_dot(agg, wg_ref[...]) + bg_ref[...])
        ni = ni_ref[...]
        h = _relu(_dot(ni, wfa_ref[...]) + _dot(hg, wfb_ref[...]) + bf_ref[...])
        if last:
            wo1a_ref, wo1b_ref, bo1_ref, wo2_ref, bo2_ref, out_ref = rest
            m = _relu(_dot(ni, wo1a_ref[...]) + _dot(h, wo1b_ref[...])
                      + bo1_ref[...])
            out_ref[...] = _relu(_dot(m, wo2_ref[...]) + bo2_ref[...])
        else:
            (unext_ref,) = rest
            unext_ref[...] = dis * h

    in_specs = [
        pl.BlockSpec((2, BR, C), lambda i: (0, i, 0)),
        pl.BlockSpec((BR, C), lambda i: (i, 0)),
        pl.BlockSpec((BR, C), lambda i: (i, 0)),
        pl.BlockSpec((2, BR, _DEGW), lambda i: (0, i, 0)),
        _full_spec(C, C),
        _full_spec(1, C),
        _full_spec(C, C),
        _full_spec(C, C),
        _full_spec(1, C),
    ]
    if last:
        in_specs += [_full_spec(C, C), _full_spec(C, C), _full_spec(1, C),
                     _full_spec(C, C), _full_spec(1, C)]
    return pl.pallas_call(
        body,
        grid=(NP // BR,),
        in_specs=in_specs,
        out_specs=pl.BlockSpec((BR, C), lambda i: (i, 0)),
        out_shape=jax.ShapeDtypeStruct((NP, C), jnp.float32),
    )


def kernel(x, edge_index, W_pre, b_pre, W_in1, b_in1, W_in2, b_in2,
           W_gcn, b_gcn, W_fc, b_fc, W_out1, b_out1, W_out2, b_out2):
    N, D = x.shape
    C = W_pre.shape[1]
    L = W_gcn.shape[0]
    E = edge_index.shape[1]

    BR = 1024
    NP = _pad_to(N + 1, max(BR, _NSUB * 8))
    Ep = _pad_to(E, _NTILES * _CHUNK * _NB)

    idt = edge_index.dtype
    src = edge_index[0]
    dst = edge_index[1]
    pe = Ep - E
    if pe:
        src = jnp.concatenate([src, jnp.zeros((pe,), idt)])
        pad_dst = N + (jnp.arange(pe, dtype=idt) % jnp.asarray(NP - N, idt))
        dst = jnp.concatenate([dst, pad_dst])
    src = src.reshape(Ep // _CHUNK, _CHUNK)
    dst = dst.reshape(Ep // _CHUNK, _CHUNK)
    xp = jnp.pad(x, ((0, NP - N), (0, 0)))
    zerosC = jnp.zeros((NP, C), jnp.float32)
    zerosW = jnp.zeros((NP, _DEGW), jnp.float32)
    onesW = jnp.ones((_CHUNK, _DEGW), jnp.float32)

    deg_parts = _deg_kernel(NP, Ep)(dst, onesW, zerosW)
    ni, h0 = _pre_kernel(NP, D, C, BR)(
        xp, W_pre, b_pre.reshape(1, C), W_in1, b_in1.reshape(1, C),
        W_in2, b_in2.reshape(1, C))
    u = _u_kernel(NP, C, BR)(h0, deg_parts)

    prop = _prop_kernel(NP, C, Ep)
    out = None
    for l in range(L):
        s_parts = prop(u, src, dst, zerosC)
        last = l == L - 1
        args = [s_parts, u, ni, deg_parts, W_gcn[l], b_gcn[l].reshape(1, C),
                W_fc[l][:C], W_fc[l][C:], b_fc[l].reshape(1, C)]
        if last:
            out = _layer_kernel(NP, C, BR, True)(
                *args, W_out1[:C], W_out1[C:], b_out1.reshape(1, C),
                W_out2, b_out2.reshape(1, C))
        else:
            u = _layer_kernel(NP, C, BR, False)(*args)
    return out[:N]


# speedup vs baseline: 2.1399x; 2.1399x over previous
import functools

import jax
import jax.numpy as jnp
from jax import lax
from jax.experimental import pallas as pl
from jax.experimental.pallas import tpu as pltpu
from jax.experimental.pallas import tpu_sc as plsc

_NCORES = 2
_NSUB = 16
_NTILES = _NCORES * _NSUB
_CHUNK = 128
_NB = 4
_DEGW = 8

_SLOW_CORE = 1
_SLOW_FRAC = 0.5


def _chunk_split(total_chunks):
    ns = int(total_chunks * _SLOW_FRAC) // (_NSUB * _NB) * _NB
    nf = total_chunks // _NSUB - ns
    return ns, nf


def _pad_to(v, m):
    return (v + m - 1) // m * m


def _relu(v):
    return jnp.maximum(v, 0.0)


def _dot(a, b):
    return jnp.dot(a, b, preferred_element_type=jnp.float32)


def _mesh():
    return plsc.VectorSubcoreMesh(core_axis_name="c", subcore_axis_name="s",
                                  num_cores=_NCORES, num_subcores=_NSUB)


@functools.lru_cache(maxsize=None)
def _deg_kernel(NP, Ep):
    ns, nf = _chunk_split(Ep // _CHUNK)
    rpt = NP // _NSUB

    @functools.partial(
        pl.kernel,
        out_type=jax.ShapeDtypeStruct((_NCORES, NP, _DEGW), jnp.float32),
        mesh=_mesh(),
        scratch_types=[
            pltpu.VMEM((nf, _CHUNK), jnp.int32),
            pltpu.VMEM((_CHUNK, _DEGW), jnp.float32),
            pltpu.VMEM_SHARED((NP, _DEGW), jnp.float32),
        ] + [pltpu.SemaphoreType.DMA] * _NB,
        compiler_params=pltpu.CompilerParams(use_tc_tiling_on_sc=False),
    )
    def deg_kernel(dst_hbm, ones_hbm, zeros_hbm, out_hbm, dst_i, ones_v,
                   acc_sh, *sems):
        c = lax.axis_index("c")
        s = lax.axis_index("s")
        nch = jnp.where(c == _SLOW_CORE, ns, nf)
        base = jnp.where(c == _SLOW_CORE, s * ns, _NSUB * ns + s * nf)
        pltpu.sync_copy(dst_hbm.at[pl.ds(base, nf)], dst_i)
        pltpu.sync_copy(ones_hbm, ones_v)
        pltpu.sync_copy(zeros_hbm.at[pl.ds(s * rpt, rpt)],
                        acc_sh.at[pl.ds(s * rpt, rpt)])
        plsc.subcore_barrier()

        for b in range(_NB):
            pltpu.async_copy(ones_v, acc_sh.at[dst_i.at[b]], sems[b],
                             add=True)

        def group(jj, carry):
            j0 = jj * _NB
            for b in range(_NB):
                j = j0 + b
                pltpu.make_async_copy(ones_v, acc_sh.at[dst_i.at[j]],
                                      sems[b]).wait()
                jn = j + _NB

                @pl.when(jn < nch)
                def _():
                    pltpu.async_copy(ones_v, acc_sh.at[dst_i.at[jn]],
                                     sems[b], add=True)
            return carry

        lax.fori_loop(0, nch // _NB, group, 0)
        plsc.subcore_barrier()
        pltpu.sync_copy(acc_sh.at[pl.ds(s * rpt, rpt)],
                        out_hbm.at[c, pl.ds(s * rpt, rpt)])

    return deg_kernel


@functools.lru_cache(maxsize=None)
def _prop_kernel(NP, C, Ep):
    ns, nf = _chunk_split(Ep // _CHUNK)
    rpt = NP // _NSUB

    @functools.partial(
        pl.kernel,
        out_type=jax.ShapeDtypeStruct((_NCORES, NP, C), jnp.float32),
        mesh=_mesh(),
        scratch_types=[
            pltpu.VMEM((nf, _CHUNK), jnp.int32),
            pltpu.VMEM((nf, _CHUNK), jnp.int32),
        ] + [pltpu.VMEM((_CHUNK, C), jnp.float32)] * _NB + [
            pltpu.VMEM_SHARED((NP, C), jnp.float32),
            pltpu.VMEM_SHARED((NP, C), jnp.float32),
        ] + [pltpu.SemaphoreType.DMA] * _NB,
        compiler_params=pltpu.CompilerParams(use_tc_tiling_on_sc=False),
    )
    def prop_kernel(u_hbm, src_hbm, dst_hbm, zeros_hbm, out_hbm,
                    src_i, dst_i, *rest):
        rows = rest[:_NB]
        acc_sh = rest[_NB]
        u_sh = rest[_NB + 1]
        sems = rest[_NB + 2:]
        c = lax.axis_index("c")
        s = lax.axis_index("s")
        nch = jnp.where(c == _SLOW_CORE, ns, nf)
        base = jnp.where(c == _SLOW_CORE, s * ns, _NSUB * ns + s * nf)
        pltpu.sync_copy(src_hbm.at[pl.ds(base, nf)], src_i)
        pltpu.sync_copy(dst_hbm.at[pl.ds(base, nf)], dst_i)
        pltpu.sync_copy(u_hbm.at[pl.ds(s * rpt, rpt)],
                        u_sh.at[pl.ds(s * rpt, rpt)])
        pltpu.sync_copy(zeros_hbm.at[pl.ds(s * rpt, rpt)],
                        acc_sh.at[pl.ds(s * rpt, rpt)])
        plsc.subcore_barrier()

        for b in range(_NB):
            pltpu.async_copy(u_sh.at[src_i.at[b]], rows[b], sems[b])

        def group(jj, carry):
            j0 = jj * _NB
            for b in range(_NB):
                j = j0 + b
                pltpu.make_async_copy(u_sh.at[src_i.at[j]], rows[b],
                                      sems[b]).wait()
                pltpu.sync_copy(rows[b], acc_sh.at[dst_i.at[j]], add=True)
                jn = j + _NB

                @pl.when(jn < nch)
                def _():
                    pltpu.async_copy(u_sh.at[src_i.at[jn]], rows[b], sems[b])
            return carry

        lax.fori_loop(0, nch // _NB, group, 0)
        plsc.subcore_barrier()
        pltpu.sync_copy(acc_sh.at[pl.ds(s * rpt, rpt)],
                        out_hbm.at[c, pl.ds(s * rpt, rpt)])

    return prop_kernel


def _full_spec(*shape):
    nd = len(shape)
    return pl.BlockSpec(shape, lambda i, _nd=nd: (0,) * _nd)


@functools.lru_cache(maxsize=None)
def _pre_kernel(NP, D, C, BR):
    def body(x_ref, wp_ref, bp_ref, w1_ref, b1_ref, w2_ref, b2_ref,
             ni_ref, h0_ref):
        h = _relu(_dot(x_ref[...], wp_ref[...]) + bp_ref[...])
        ni_ref[...] = _relu(_dot(h, w1_ref[...]) + b1_ref[...])
        h0_ref[...] = _relu(_dot(h, w2_ref[...]) + b2_ref[...])

    return pl.pallas_call(
        body,
        grid=(NP // BR,),
        in_specs=[
            pl.BlockSpec((BR, D), lambda i: (i, 0)),
            _full_spec(D, C),
            _full_spec(1, C),
            _full_spec(C, C),
            _full_spec(1, C),
            _full_spec(C, C),
            _full_spec(1, C),
        ],
        out_specs=[pl.BlockSpec((BR, C), lambda i: (i, 0)),
                   pl.BlockSpec((BR, C), lambda i: (i, 0))],
        out_shape=[jax.ShapeDtypeStruct((NP, C), jnp.float32),
                   jax.ShapeDtypeStruct((NP, C), jnp.float32)],
    )


@functools.lru_cache(maxsize=None)
def _u_kernel(NP, C, BR):
    def body(h_ref, dp_ref, u_ref):
        dpa = dp_ref[...]
        dis = lax.rsqrt(dpa[0, :, 0:1] + dpa[1, :, 0:1] + 1.0)
        u_ref[...] = dis * h_ref[...]

    return pl.pallas_call(
        body,
        grid=(NP // BR,),
        in_specs=[
            pl.BlockSpec((BR, C), lambda i: (i, 0)),
            pl.BlockSpec((2, BR, _DEGW), lambda i: (0, i, 0)),
        ],
        out_specs=pl.BlockSpec((BR, C), lambda i: (i, 0)),
        out_shape=jax.ShapeDtypeStruct((NP, C), jnp.float32),
    )


@functools.lru_cache(maxsize=None)
def _layer_kernel(NP, C, BR, last):
    def body(sp_ref, u_ref, ni_ref, dp_ref, wg_ref, bg_ref, wfa_ref, wfb_ref,
             bf_ref, *rest):
        dpa = dp_ref[...]
        dis = lax.rsqrt(dpa[0, :, 0:1] + dpa[1, :, 0:1] + 1.0)
        spa = sp_ref[...]
        u = u_ref[...]
        agg = dis * (spa[0] + spa[1] + u)
        hg = _relu(_dot(agg, wg_ref[...]) + bg_ref[...])
        ni = ni_ref[...]
        h = _relu(_dot(ni, wfa_ref[...]) + _dot(hg, wfb_ref[...]) + bf_ref[...])
        if last:
            wo1a_ref, wo1b_ref, bo1_ref, wo2_ref, bo2_ref, out_ref = rest
            m = _relu(_dot(ni, wo1a_ref[...]) + _dot(h, wo1b_ref[...])
                      + bo1_ref[...])
            out_ref[...] = _relu(_dot(m, wo2_ref[...]) + bo2_ref[...])
        else:
            (unext_ref,) = rest
            unext_ref[...] = dis * h

    in_specs = [
        pl.BlockSpec((2, BR, C), lambda i: (0, i, 0)),
        pl.BlockSpec((BR, C), lambda i: (i, 0)),
        pl.BlockSpec((BR, C), lambda i: (i, 0)),
        pl.BlockSpec((2, BR, _DEGW), lambda i: (0, i, 0)),
        _full_spec(C, C),
        _full_spec(1, C),
        _full_spec(C, C),
        _full_spec(C, C),
        _full_spec(1, C),
    ]
    if last:
        in_specs += [_full_spec(C, C), _full_spec(C, C), _full_spec(1, C),
                     _full_spec(C, C), _full_spec(1, C)]
    return pl.pallas_call(
        body,
        grid=(NP // BR,),
        in_specs=in_specs,
        out_specs=pl.BlockSpec((BR, C), lambda i: (i, 0)),
        out_shape=jax.ShapeDtypeStruct((NP, C), jnp.float32),
    )


def kernel(x, edge_index, W_pre, b_pre, W_in1, b_in1, W_in2, b_in2,
           W_gcn, b_gcn, W_fc, b_fc, W_out1, b_out1, W_out2, b_out2):
    N, D = x.shape
    C = W_pre.shape[1]
    L = W_gcn.shape[0]
    E = edge_index.shape[1]

    BR = 1024
    NP = _pad_to(N + 1, max(BR, _NSUB * 8))
    Ep = _pad_to(E, _NTILES * _CHUNK * _NB)

    idt = edge_index.dtype
    src = edge_index[0]
    dst = edge_index[1]
    pe = Ep - E
    if pe:
        src = jnp.concatenate([src, jnp.zeros((pe,), idt)])
        pad_dst = N + (jnp.arange(pe, dtype=idt) % jnp.asarray(NP - N, idt))
        dst = jnp.concatenate([dst, pad_dst])
    src = src.reshape(Ep // _CHUNK, _CHUNK)
    dst = dst.reshape(Ep // _CHUNK, _CHUNK)
    xp = jnp.pad(x, ((0, NP - N), (0, 0)))
    zerosC = jnp.zeros((NP, C), jnp.float32)
    zerosW = jnp.zeros((NP, _DEGW), jnp.float32)
    onesW = jnp.ones((_CHUNK, _DEGW), jnp.float32)

    deg_parts = _deg_kernel(NP, Ep)(dst, onesW, zerosW)
    ni, h0 = _pre_kernel(NP, D, C, BR)(
        xp, W_pre, b_pre.reshape(1, C), W_in1, b_in1.reshape(1, C),
        W_in2, b_in2.reshape(1, C))
    u = _u_kernel(NP, C, BR)(h0, deg_parts)

    prop = _prop_kernel(NP, C, Ep)
    out = None
    for l in range(L):
        s_parts = prop(u, src, dst, zerosC)
        last = l == L - 1
        args = [s_parts, u, ni, deg_parts, W_gcn[l], b_gcn[l].reshape(1, C),
                W_fc[l][:C], W_fc[l][C:], b_fc[l].reshape(1, C)]
        if last:
            out = _layer_kernel(NP, C, BR, True)(
                *args, W_out1[:C], W_out1[C:], b_out1.reshape(1, C),
                W_out2, b_out2.reshape(1, C))
        else:
            u = _layer_kernel(NP, C, BR, False)(*args)
    return out[:N]
